# baseline (speedup 1.0000x reference)
import functools

import jax
import jax.numpy as jnp
from jax import lax
from jax.experimental import pallas as pl
from jax.experimental.pallas import tpu as pltpu
from jax.experimental.pallas import tpu_sc as plsc

N = 10000
D = 128
H = 128
C = 10
E = 320000

NC = 2
NS = 16
NW = NC * NS

NPAD = 10240
RPT = NPAD // NS
F = 16
CH = 128
EW = 10240
NCHUNK = EW // CH
EPAD = EW * NW
PAD_NODE = N

_mesh = plsc.VectorSubcoreMesh(core_axis_name="c", subcore_axis_name="s",
                               num_cores=1)
_sc_params = pltpu.CompilerParams(use_tc_tiling_on_sc=False)


KG = 16
NCHT = EPAD // CH
NCHW = NCHT // NS
NGW = NCHW // KG


@functools.partial(
    pl.kernel,
    out_type=jax.ShapeDtypeStruct((NPAD,), jnp.float32),
    mesh=_mesh,
    compiler_params=_sc_params,
    scratch_types=[
        pltpu.VMEM((KG, 2, CH), jnp.int32),
        pltpu.VMEM((CH,), jnp.float32),
        pltpu.VMEM((RPT,), jnp.float32),
        pltpu.VMEM_SHARED((NPAD,), jnp.float32),
        pltpu.SemaphoreType.DMA,
        pltpu.SemaphoreType.DMA,
    ],
)
def _deg_kernel(idx_hbm, degp_hbm, ibuf, ones, zb, acc,
                sem_i, sem_s):
    s = lax.axis_index("s")
    cbase = s * NCHW
    for i in range(CH // 16):
        ones[pl.ds(i * 16, 16)] = jnp.ones((16,), jnp.float32)

    def zrow(i, carry):
        zb[pl.ds(i * 16, 16)] = jnp.zeros((16,), jnp.float32)
        return carry

    lax.fori_loop(0, RPT // 16, zrow, 0)
    pltpu.sync_copy(zb, acc.at[pl.ds(s * RPT, RPT)])
    plsc.subcore_barrier()

    def group(g, carry):
        di = [pltpu.async_copy(idx_hbm.at[cbase + g * KG + b], ibuf.at[b],
                               sem_i)
              for b in range(KG)]
        ds = []
        for b in range(KG):
            di[b].wait()
            ds.append(pltpu.async_copy(ones, acc.at[ibuf.at[b, 1]], sem_s,
                                       add=True))
        for d in ds:
            d.wait()
        return carry

    lax.fori_loop(0, NGW, group, 0)
    plsc.subcore_barrier()
    pltpu.sync_copy(acc.at[pl.ds(s * RPT, RPT)],
                    degp_hbm.at[pl.ds(s * RPT, RPT)])


@functools.partial(
    pl.kernel,
    out_type=jax.ShapeDtypeStruct((NPAD, F), jnp.float32),
    mesh=_mesh,
    compiler_params=_sc_params,
    scratch_types=[
        pltpu.VMEM((KG, 2, CH), jnp.int32),
        pltpu.VMEM((KG, CH, F), jnp.float32),
        pltpu.VMEM((RPT, F), jnp.float32),
        pltpu.VMEM_SHARED((NPAD, F), jnp.float32),
        pltpu.SemaphoreType.DMA,
        pltpu.SemaphoreType.DMA,
        pltpu.SemaphoreType.DMA,
    ],
)
def _hop_kernel(u_hbm, idx_hbm, out_hbm,
                ibuf, rows, zb, acc, sem_i, sem_g, sem_s):
    s = lax.axis_index("s")
    cbase = s * NCHW

    def zrow(i, carry):
        for r in range(8):
            zb[i * 8 + r] = jnp.zeros((16,), jnp.float32)
        return carry

    lax.fori_loop(0, RPT // 8, zrow, 0)
    pltpu.sync_copy(zb, acc.at[pl.ds(s * RPT, RPT)])
    plsc.subcore_barrier()

    def group(g, carry):
        di = [pltpu.async_copy(idx_hbm.at[cbase + g * KG + b], ibuf.at[b],
                               sem_i)
              for b in range(KG)]
        dg = []
        for b in range(KG):
            di[b].wait()
            dg.append(pltpu.async_copy(u_hbm.at[ibuf.at[b, 0]], rows.at[b],
                                       sem_g))
        ds = []
        for b in range(KG):
            dg[b].wait()
            ds.append(pltpu.async_copy(rows.at[b], acc.at[ibuf.at[b, 1]],
                                       sem_s, add=True))
        for d in ds:
            d.wait()
        return carry

    lax.fori_loop(0, NGW, group, 0)
    plsc.subcore_barrier()
    pltpu.sync_copy(acc.at[pl.ds(s * RPT, RPT)],
                    out_hbm.at[pl.ds(s * RPT, RPT)])


def _proj_body(x_ref, wcat_ref, bcat_ref, w1r_ref, degp_ref, u_ref):
    h = jnp.dot(x_ref[...], wcat_ref[...], preferred_element_type=jnp.float32)
    h = jnp.maximum(h + bcat_ref[...], 0.0)
    z = jnp.dot(h, w1r_ref[...], preferred_element_type=jnp.float32)
    deg = jnp.maximum(degp_ref[0], 1.0)
    u_ref[...] = z * lax.rsqrt(deg)[:, None]


def _proj(xp, wcat, bcat, w1r, degp):
    blk = NPAD // 4
    return pl.pallas_call(
        _proj_body,
        grid=(4,),
        in_specs=[
            pl.BlockSpec((blk, D), lambda i: (i, 0)),
            pl.BlockSpec((D, 3 * H), lambda i: (0, 0)),
            pl.BlockSpec((1, 3 * H), lambda i: (0, 0)),
            pl.BlockSpec((3 * H, F), lambda i: (0, 0)),
            pl.BlockSpec((1, blk), lambda i: (0, i)),
        ],
        out_specs=pl.BlockSpec((blk, F), lambda i: (i, 0)),
        out_shape=jax.ShapeDtypeStruct((NPAD, F), jnp.float32),
    )(xp, wcat, bcat, w1r, degp)


def _scale_inv_body(wp_ref, degp_ref, v_ref):
    deg = jnp.maximum(degp_ref[0], 1.0)
    v_ref[...] = wp_ref[...] / deg[:, None]


def _final_body(yp_ref, degp_ref, b1_ref, out_ref):
    deg = jnp.maximum(degp_ref[0], 1.0)
    y = yp_ref[...] * (lax.rsqrt(deg) * (1.0 / 3.0))[:, None]
    logits = (y + b1_ref[...])[:, :C]
    m = jnp.max(logits, axis=1, keepdims=True)
    lse = m + jnp.log(jnp.sum(jnp.exp(logits - m), axis=1, keepdims=True))
    out_ref[...] = (logits - lse)[:N]


def kernel(x, edge_index, W11, b11, W12, b12, W13, b13, W1, b1):
    pad_e = jnp.full((2, EPAD - E), PAD_NODE, jnp.int32)
    eip = jnp.concatenate([edge_index, pad_e], axis=1)
    ei_il = jnp.transpose(eip.reshape(2, NCHT, CH), (1, 0, 2))
    xp = jnp.pad(x, ((0, NPAD - N), (0, 0)))
    wcat = jnp.concatenate([W11, W12, W13], axis=1)
    bcat = jnp.concatenate([b11, b12, b13]).reshape(1, 3 * H)
    w1p = jnp.pad(W1, ((0, 0), (0, F - C)))
    w1r = jnp.concatenate([w1p, w1p, w1p], axis=0)
    b1p = jnp.pad(b1, (0, F - C)).reshape(1, F)

    degp = _deg_kernel(ei_il).reshape(1, NPAD)
    u = _proj(xp, wcat, bcat, w1r, degp)

    wp = _hop_kernel(u, ei_il)

    v = pl.pallas_call(
        _scale_inv_body,
        out_shape=jax.ShapeDtypeStruct((NPAD, F), jnp.float32),
    )(wp, degp)

    yp = _hop_kernel(v, ei_il)

    out = pl.pallas_call(
        _final_body,
        out_shape=jax.ShapeDtypeStruct((N, C), jnp.float32),
    )(yp, degp, b1p)
    return out


# speedup vs baseline: 1.2009x; 1.2009x over previous
import functools

import jax
import jax.numpy as jnp
from jax import lax
from jax.experimental import pallas as pl
from jax.experimental.pallas import tpu as pltpu
from jax.experimental.pallas import tpu_sc as plsc

N = 10000
D = 128
H = 128
C = 10
E = 320000

NC = 2
NS = 16
NW = NC * NS

NPAD = 10240
RPT = NPAD // NS
F = 16
CH = 128
EW = 10240
NCHUNK = EW // CH
EPAD = EW * NW
PAD_NODE = N

_mesh = plsc.VectorSubcoreMesh(core_axis_name="c", subcore_axis_name="s")
_sc_params = pltpu.CompilerParams(use_tc_tiling_on_sc=False)


KG = 16
NCHT = EPAD // CH
NCH0 = 144
NCH1 = (NCHT - NCH0 * NS) // NS
NG0 = NCH0 // KG
NG1 = NCH1 // KG


@functools.partial(
    pl.kernel,
    out_type=jax.ShapeDtypeStruct((NC, NPAD), jnp.float32),
    mesh=_mesh,
    compiler_params=_sc_params,
    scratch_types=[
        pltpu.VMEM((KG, 2, CH), jnp.int32),
        pltpu.VMEM((CH,), jnp.float32),
        pltpu.VMEM((RPT,), jnp.float32),
        pltpu.VMEM_SHARED((NPAD,), jnp.float32),
        pltpu.SemaphoreType.DMA,
        pltpu.SemaphoreType.DMA,
    ],
)
def _deg_kernel(idx_hbm, degp_hbm, ibuf, ones, zb, acc,
                sem_i, sem_s):
    c = lax.axis_index("c")
    s = lax.axis_index("s")
    cbase = jnp.where(c == 0, s * NCH0, NCH0 * NS + s * NCH1)
    ng = jnp.where(c == 0, NG0, NG1)
    for i in range(CH // 16):
        ones[pl.ds(i * 16, 16)] = jnp.ones((16,), jnp.float32)

    def zrow(i, carry):
        zb[pl.ds(i * 16, 16)] = jnp.zeros((16,), jnp.float32)
        return carry

    lax.fori_loop(0, RPT // 16, zrow, 0)
    pltpu.sync_copy(zb, acc.at[pl.ds(s * RPT, RPT)])
    plsc.subcore_barrier()

    def group(g, carry):
        di = [pltpu.async_copy(idx_hbm.at[cbase + g * KG + b], ibuf.at[b],
                               sem_i)
              for b in range(KG)]
        ds = []
        for b in range(KG):
            di[b].wait()
            ds.append(pltpu.async_copy(ones, acc.at[ibuf.at[b, 1]], sem_s,
                                       add=True))
        for d in ds:
            d.wait()
        return carry

    lax.fori_loop(0, ng, group, 0)
    plsc.subcore_barrier()
    pltpu.sync_copy(acc.at[pl.ds(s * RPT, RPT)],
                    degp_hbm.at[c, pl.ds(s * RPT, RPT)])


@functools.partial(
    pl.kernel,
    out_type=jax.ShapeDtypeStruct((NC, NPAD, F), jnp.float32),
    mesh=_mesh,
    compiler_params=_sc_params,
    scratch_types=[
        pltpu.VMEM((KG, 2, CH), jnp.int32),
        pltpu.VMEM((KG, CH, F), jnp.float32),
        pltpu.VMEM((RPT, F), jnp.float32),
        pltpu.VMEM_SHARED((NPAD, F), jnp.float32),
        pltpu.SemaphoreType.DMA,
        pltpu.SemaphoreType.DMA,
        pltpu.SemaphoreType.DMA,
    ],
)
def _hop_kernel(u_hbm, idx_hbm, out_hbm,
                ibuf, rows, zb, acc, sem_i, sem_g, sem_s):
    c = lax.axis_index("c")
    s = lax.axis_index("s")
    cbase = jnp.where(c == 0, s * NCH0, NCH0 * NS + s * NCH1)
    ng = jnp.where(c == 0, NG0, NG1)

    def zrow(i, carry):
        for r in range(8):
            zb[i * 8 + r] = jnp.zeros((16,), jnp.float32)
        return carry

    lax.fori_loop(0, RPT // 8, zrow, 0)
    pltpu.sync_copy(zb, acc.at[pl.ds(s * RPT, RPT)])
    plsc.subcore_barrier()

    def group(g, carry):
        di = [pltpu.async_copy(idx_hbm.at[cbase + g * KG + b], ibuf.at[b],
                               sem_i)
              for b in range(KG)]
        dg = []
        for b in range(KG):
            di[b].wait()
            dg.append(pltpu.async_copy(u_hbm.at[ibuf.at[b, 0]], rows.at[b],
                                       sem_g))
        ds = []
        for b in range(KG):
            dg[b].wait()
            ds.append(pltpu.async_copy(rows.at[b], acc.at[ibuf.at[b, 1]],
                                       sem_s, add=True))
        for d in ds:
            d.wait()
        return carry

    lax.fori_loop(0, ng, group, 0)
    plsc.subcore_barrier()
    pltpu.sync_copy(acc.at[pl.ds(s * RPT, RPT)],
                    out_hbm.at[c, pl.ds(s * RPT, RPT)])


def _proj_body(x_ref, wcat_ref, bcat_ref, w1r_ref, degpt_ref, u_ref):
    h = jnp.dot(x_ref[...], wcat_ref[...], preferred_element_type=jnp.float32)
    h = jnp.maximum(h + bcat_ref[...], 0.0)
    z = jnp.dot(h, w1r_ref[...], preferred_element_type=jnp.float32)
    deg = jnp.maximum(degpt_ref[:, 0] + degpt_ref[:, 1], 1.0)
    u_ref[...] = z * lax.rsqrt(deg)[:, None]


def _proj(x, wcat, bcat, w1r, degp):
    blk = N // 5
    return pl.pallas_call(
        _proj_body,
        grid=(5,),
        in_specs=[
            pl.BlockSpec((blk, D), lambda i: (i, 0)),
            pl.BlockSpec((D, 3 * H), lambda i: (0, 0)),
            pl.BlockSpec((1, 3 * H), lambda i: (0, 0)),
            pl.BlockSpec((3 * H, F), lambda i: (0, 0)),
            pl.BlockSpec((blk, NC), lambda i: (i, 0)),
        ],
        out_specs=pl.BlockSpec((blk, F), lambda i: (i, 0)),
        out_shape=jax.ShapeDtypeStruct((N, F), jnp.float32),
    )(x, wcat, bcat, w1r, degp)


def _scale_inv_body(wp_ref, degp_ref, v_ref):
    deg = jnp.maximum(degp_ref[0] + degp_ref[1], 1.0)
    v_ref[...] = (wp_ref[0] + wp_ref[1]) / deg[:, None]


def _final_body(yp_ref, degp_ref, b1_ref, out_ref):
    deg = jnp.maximum(degp_ref[0] + degp_ref[1], 1.0)
    y = (yp_ref[0] + yp_ref[1]) * (lax.rsqrt(deg) * (1.0 / 3.0))[:, None]
    logits = (y + b1_ref[...])[:, :C]
    m = jnp.max(logits, axis=1, keepdims=True)
    lse = m + jnp.log(jnp.sum(jnp.exp(logits - m), axis=1, keepdims=True))
    out_ref[...] = (logits - lse)[:N]


def kernel(x, edge_index, W11, b11, W12, b12, W13, b13, W1, b1):
    pad_e = jnp.full((2, EPAD - E), PAD_NODE, jnp.int32)
    eip = jnp.concatenate([edge_index, pad_e], axis=1)
    ei_il = jnp.transpose(eip.reshape(2, NCHT, CH), (1, 0, 2))
    wcat = jnp.concatenate([W11, W12, W13], axis=1)
    bcat = jnp.concatenate([b11, b12, b13]).reshape(1, 3 * H)
    w1p = jnp.pad(W1, ((0, 0), (0, F - C)))
    w1r = jnp.concatenate([w1p, w1p, w1p], axis=0)
    b1p = jnp.pad(b1, (0, F - C)).reshape(1, F)

    degp = _deg_kernel(ei_il)
    u = jnp.pad(_proj(x, wcat, bcat, w1r, degp[:, :N].T),
                ((0, NPAD - N), (0, 0)))

    wp = _hop_kernel(u, ei_il)

    v = pl.pallas_call(
        _scale_inv_body,
        out_shape=jax.ShapeDtypeStruct((NPAD, F), jnp.float32),
    )(wp, degp)

    yp = _hop_kernel(v, ei_il)

    out = pl.pallas_call(
        _final_body,
        out_shape=jax.ShapeDtypeStruct((N, C), jnp.float32),
    )(yp, degp, b1p)
    return out


# speedup vs baseline: 1.2653x; 1.0536x over previous
import functools

import jax
import jax.numpy as jnp
from jax import lax
from jax.experimental import pallas as pl
from jax.experimental.pallas import tpu as pltpu
from jax.experimental.pallas import tpu_sc as plsc

N = 10000
D = 128
H = 128
C = 10
E = 320000

NC = 2
NS = 16
NW = NC * NS

NPAD = 10240
RPT = NPAD // NS
F = 16
CH = 128
EW = 10240
NCHUNK = EW // CH
EPAD = EW * NW
PAD_NODE = N

_mesh = plsc.VectorSubcoreMesh(core_axis_name="c", subcore_axis_name="s")
_sc_params = pltpu.CompilerParams(use_tc_tiling_on_sc=False)


KG = 16
NCHT = EPAD // CH
NCH0 = 144
NCH1 = (NCHT - NCH0 * NS) // NS
NG0 = NCH0 // KG
NG1 = NCH1 // KG


@functools.partial(
    pl.kernel,
    out_type=jax.ShapeDtypeStruct((NC, NPAD), jnp.float32),
    mesh=_mesh,
    compiler_params=_sc_params,
    scratch_types=[
        pltpu.VMEM((KG, 2, CH), jnp.int32),
        pltpu.VMEM((CH,), jnp.float32),
        pltpu.VMEM((RPT,), jnp.float32),
        pltpu.VMEM_SHARED((NPAD,), jnp.float32),
        pltpu.SemaphoreType.DMA,
        pltpu.SemaphoreType.DMA,
    ],
)
def _deg_kernel(idx_hbm, degp_hbm, ibuf, ones, zb, acc,
                sem_i, sem_s):
    c = lax.axis_index("c")
    s = lax.axis_index("s")
    cbase = jnp.where(c == 0, s * NCH0, NCH0 * NS + s * NCH1)
    ng = jnp.where(c == 0, NG0, NG1)
    for i in range(CH // 16):
        ones[pl.ds(i * 16, 16)] = jnp.ones((16,), jnp.float32)

    def zrow(i, carry):
        zb[pl.ds(i * 16, 16)] = jnp.zeros((16,), jnp.float32)
        return carry

    lax.fori_loop(0, RPT // 16, zrow, 0)
    pltpu.sync_copy(zb, acc.at[pl.ds(s * RPT, RPT)])
    plsc.subcore_barrier()

    def group(g, carry):
        di = [pltpu.async_copy(idx_hbm.at[cbase + g * KG + b], ibuf.at[b],
                               sem_i)
              for b in range(KG)]
        ds = []
        for b in range(KG):
            di[b].wait()
            ds.append(pltpu.async_copy(ones, acc.at[ibuf.at[b, 1]], sem_s,
                                       add=True))
        for d in ds:
            d.wait()
        return carry

    lax.fori_loop(0, ng, group, 0)
    plsc.subcore_barrier()
    pltpu.sync_copy(acc.at[pl.ds(s * RPT, RPT)],
                    degp_hbm.at[c, pl.ds(s * RPT, RPT)])


@functools.partial(
    pl.kernel,
    out_type=jax.ShapeDtypeStruct((NC, NPAD, F), jnp.float32),
    mesh=_mesh,
    compiler_params=_sc_params,
    scratch_types=[
        pltpu.VMEM((KG, 2, CH), jnp.int32),
        pltpu.VMEM((KG, CH, F), jnp.float32),
        pltpu.VMEM((RPT, F), jnp.float32),
        pltpu.VMEM((RPT, F), jnp.float32),
        pltpu.VMEM_SHARED((NPAD, F), jnp.float32),
        pltpu.VMEM_SHARED((NPAD, F), jnp.float32),
        pltpu.SemaphoreType.DMA,
        pltpu.SemaphoreType.DMA,
        pltpu.SemaphoreType.DMA,
    ],
)
def _hop_kernel(u_hbm, idx_hbm, out_hbm,
                ibuf, rows, zb, zb2, acc, vsrc, sem_i, sem_g, sem_s):
    c = lax.axis_index("c")
    s = lax.axis_index("s")

    def zrow(i, carry):
        for r in range(8):
            zb[i * 8 + r] = jnp.zeros((16,), jnp.float32)
        return carry

    lax.fori_loop(0, RPT // 8, zrow, 0)
    pltpu.sync_copy(zb, acc.at[pl.ds(s * RPT, RPT)])

    @pl.when(c == 0)
    def _stage():
        pltpu.sync_copy(u_hbm.at[pl.ds(s * RPT, RPT)], zb2)
        pltpu.sync_copy(zb2, vsrc.at[pl.ds(s * RPT, RPT)])

    plsc.subcore_barrier()

    def run_groups(src_ref, cbase, ng):
        def group(g, carry):
            di = [pltpu.async_copy(idx_hbm.at[cbase + g * KG + b], ibuf.at[b],
                                   sem_i)
                  for b in range(KG)]
            dg = []
            for b in range(KG):
                di[b].wait()
                dg.append(pltpu.async_copy(src_ref.at[ibuf.at[b, 0]],
                                           rows.at[b], sem_g))
            ds = []
            for b in range(KG):
                dg[b].wait()
                ds.append(pltpu.async_copy(rows.at[b], acc.at[ibuf.at[b, 1]],
                                           sem_s, add=True))
            for d in ds:
                d.wait()
            return carry

        lax.fori_loop(0, ng, group, 0)

    @pl.when(c == 0)
    def _core0():
        run_groups(vsrc, s * NCH0, NG0)

    @pl.when(c != 0)
    def _core1():
        run_groups(u_hbm, NCH0 * NS + s * NCH1, NG1)

    plsc.subcore_barrier()
    pltpu.sync_copy(acc.at[pl.ds(s * RPT, RPT)],
                    out_hbm.at[c, pl.ds(s * RPT, RPT)])


def _proj_body(x_ref, wcat_ref, bcat_ref, w1r_ref, degpt_ref, u_ref):
    h = jnp.dot(x_ref[...], wcat_ref[...], preferred_element_type=jnp.float32)
    h = jnp.maximum(h + bcat_ref[...], 0.0)
    z = jnp.dot(h, w1r_ref[...], preferred_element_type=jnp.float32)
    deg = jnp.maximum(degpt_ref[:, 0] + degpt_ref[:, 1], 1.0)
    u_ref[...] = z * lax.rsqrt(deg)[:, None]


def _proj(x, wcat, bcat, w1r, degp):
    blk = N // 5
    return pl.pallas_call(
        _proj_body,
        grid=(5,),
        in_specs=[
            pl.BlockSpec((blk, D), lambda i: (i, 0)),
            pl.BlockSpec((D, 3 * H), lambda i: (0, 0)),
            pl.BlockSpec((1, 3 * H), lambda i: (0, 0)),
            pl.BlockSpec((3 * H, F), lambda i: (0, 0)),
            pl.BlockSpec((blk, NC), lambda i: (i, 0)),
        ],
        out_specs=pl.BlockSpec((blk, F), lambda i: (i, 0)),
        out_shape=jax.ShapeDtypeStruct((N, F), jnp.float32),
    )(x, wcat, bcat, w1r, degp)


def _scale_inv_body(wp_ref, degp_ref, v_ref):
    deg = jnp.maximum(degp_ref[0] + degp_ref[1], 1.0)
    v_ref[...] = (wp_ref[0] + wp_ref[1]) / deg[:, None]


def _final_body(yp_ref, degp_ref, b1_ref, out_ref):
    deg = jnp.maximum(degp_ref[0] + degp_ref[1], 1.0)
    y = (yp_ref[0] + yp_ref[1]) * (lax.rsqrt(deg) * (1.0 / 3.0))[:, None]
    logits = (y + b1_ref[...])[:, :C]
    m = jnp.max(logits, axis=1, keepdims=True)
    lse = m + jnp.log(jnp.sum(jnp.exp(logits - m), axis=1, keepdims=True))
    out_ref[...] = (logits - lse)[:N]


def kernel(x, edge_index, W11, b11, W12, b12, W13, b13, W1, b1):
    pad_e = jnp.full((2, EPAD - E), PAD_NODE, jnp.int32)
    eip = jnp.concatenate([edge_index, pad_e], axis=1)
    ei_il = jnp.transpose(eip.reshape(2, NCHT, CH), (1, 0, 2))
    wcat = jnp.concatenate([W11, W12, W13], axis=1)
    bcat = jnp.concatenate([b11, b12, b13]).reshape(1, 3 * H)
    w1p = jnp.pad(W1, ((0, 0), (0, F - C)))
    w1r = jnp.concatenate([w1p, w1p, w1p], axis=0)
    b1p = jnp.pad(b1, (0, F - C)).reshape(1, F)

    degp = _deg_kernel(ei_il)
    u = jnp.pad(_proj(x, wcat, bcat, w1r, degp[:, :N].T),
                ((0, NPAD - N), (0, 0)))

    wp = _hop_kernel(u, ei_il)

    v = pl.pallas_call(
        _scale_inv_body,
        out_shape=jax.ShapeDtypeStruct((NPAD, F), jnp.float32),
    )(wp, degp)

    yp = _hop_kernel(v, ei_il)

    out = pl.pallas_call(
        _final_body,
        out_shape=jax.ShapeDtypeStruct((N, C), jnp.float32),
    )(yp, degp, b1p)
    return out
